# baseline (speedup 1.0000x reference)
import functools

import jax
import jax.numpy as jnp
from jax import lax
from jax.experimental import pallas as pl
from jax.experimental.pallas import tpu as pltpu
from jax.experimental.pallas import tpu_sc as plsc

N_NODES = 10000
IN_DIM = 128
HID_DIM = 128
OUT_DIM = 16

NC = 2
NS = 16
NW = NC * NS
K = 128
STRIPE = 640
NPAD = NS * STRIPE
BR = 1000
NBUF = 4


def _fill(ref, rows, cols, value):
    groups = cols // 16

    def body(t, carry):
        ref[t // groups, pl.ds((t % groups) * 16, 16)] = jnp.full(
            (16,), value, jnp.float32)
        return carry

    lax.fori_loop(0, rows * groups, body, 0)


def _zero_stripe(bufs, acc, s):
    d = bufs.shape[2]
    groups = d // 16

    def body(t, carry):
        bufs[0, t // groups, pl.ds((t % groups) * 16, 16)] = jnp.zeros(
            (16,), jnp.float32)
        return carry

    lax.fori_loop(0, K * groups, body, 0)
    for kk in range(STRIPE // K):
        pltpu.sync_copy(bufs.at[0], acc.at[pl.ds(s * STRIPE + kk * K, K)])


def _drain_stripe(acc, out_hbm, c, s):
    for kk in range(STRIPE // K):
        off = s * STRIPE + kk * K
        pltpu.sync_copy(acc.at[pl.ds(off, K)], out_hbm.at[c, pl.ds(off, K)])


def _edge_pipeline(z_hbm, src_v, dst_v, bufs, gsem, acc, n_blocks):
    for b in range(NBUF):
        pltpu.async_copy(z_hbm.at[src_v.at[b]], bufs.at[b], gsem[b])

    def body(g, carry):
        for b in range(NBUF):
            j = g * NBUF + b
            pltpu.make_async_copy(
                z_hbm.at[src_v.at[j]], bufs.at[b], gsem[b]).wait()
            pltpu.sync_copy(bufs.at[b], acc.at[dst_v.at[j]], add=True)

            @pl.when(j + NBUF < n_blocks)
            def _():
                pltpu.async_copy(
                    z_hbm.at[src_v.at[j + NBUF]], bufs.at[b], gsem[b])
        return carry

    lax.fori_loop(0, n_blocks // NBUF, body, 0)


def _make_deg_kernel(n_blocks):
    mesh = plsc.VectorSubcoreMesh(core_axis_name="c", subcore_axis_name="s")

    @functools.partial(
        pl.kernel,
        mesh=mesh,
        out_type=jax.ShapeDtypeStruct((NC, NPAD, 16), jnp.float32),
        scratch_types=[
            pltpu.VMEM((n_blocks, K), jnp.int32),
            pltpu.VMEM((K, 16), jnp.float32),
            pltpu.VMEM_SHARED((NPAD, 16), jnp.float32),
        ],
        compiler_params=pltpu.CompilerParams(use_tc_tiling_on_sc=False),
    )
    def deg_kernel(dst_hbm, out_hbm, dst_v, ones_v, acc):
        c = lax.axis_index("c")
        s = lax.axis_index("s")
        wid = c * NS + s
        _fill(ones_v, K, 16, 0.0)
        for kk in range(STRIPE // K):
            pltpu.sync_copy(ones_v, acc.at[pl.ds(s * STRIPE + kk * K, K)])
        _fill(ones_v, K, 16, 1.0)
        pltpu.sync_copy(dst_hbm.at[wid], dst_v)
        plsc.subcore_barrier()

        def body(j, carry):
            pltpu.sync_copy(ones_v, acc.at[dst_v.at[j]], add=True)
            return carry

        lax.fori_loop(0, n_blocks, body, 0)
        plsc.subcore_barrier()
        _drain_stripe(acc, out_hbm, c, s)

    return deg_kernel


def _make_agg_split_kernel(n_blocks, d_half):
    assert n_blocks % NBUF == 0
    mesh = plsc.VectorSubcoreMesh(core_axis_name="c", subcore_axis_name="s")

    @functools.partial(
        pl.kernel,
        mesh=mesh,
        out_type=jax.ShapeDtypeStruct((NC, NPAD, d_half), jnp.float32),
        scratch_types=[
            pltpu.VMEM((n_blocks, K), jnp.int32),
            pltpu.VMEM((n_blocks, K), jnp.int32),
            pltpu.VMEM((NBUF, K, d_half), jnp.float32),
            pltpu.VMEM_SHARED((NPAD, d_half), jnp.float32),
        ] + [pltpu.SemaphoreType.DMA] * NBUF,
        compiler_params=pltpu.CompilerParams(use_tc_tiling_on_sc=False),
    )
    def agg_kernel(z_hbm, src_hbm, dst_hbm, out_hbm, src_v, dst_v, bufs, acc,
                   *gsem):
        c = lax.axis_index("c")
        s = lax.axis_index("s")
        _zero_stripe(bufs, acc, s)
        pltpu.sync_copy(src_hbm.at[s], src_v)
        pltpu.sync_copy(dst_hbm.at[s], dst_v)
        two = jnp.full((16,), 2, jnp.int32)

        def xform(t, carry):
            r = t // (K // 16)
            o = (t % (K // 16)) * 16
            src_v[r, pl.ds(o, 16)] = src_v[r, pl.ds(o, 16)] * two + c
            return carry

        lax.fori_loop(0, n_blocks * (K // 16), xform, 0)
        plsc.subcore_barrier()
        _edge_pipeline(z_hbm, src_v, dst_v, bufs, gsem, acc, n_blocks)
        plsc.subcore_barrier()
        _drain_stripe(acc, out_hbm, c, s)

    return agg_kernel


def _make_agg_kernel(n_blocks, d):
    assert n_blocks % NBUF == 0
    mesh = plsc.VectorSubcoreMesh(core_axis_name="c", subcore_axis_name="s")

    @functools.partial(
        pl.kernel,
        mesh=mesh,
        out_type=jax.ShapeDtypeStruct((NC, NPAD, d), jnp.float32),
        scratch_types=[
            pltpu.VMEM((n_blocks, K), jnp.int32),
            pltpu.VMEM((n_blocks, K), jnp.int32),
            pltpu.VMEM((NBUF, K, d), jnp.float32),
            pltpu.VMEM_SHARED((NPAD, d), jnp.float32),
        ] + [pltpu.SemaphoreType.DMA] * NBUF,
        compiler_params=pltpu.CompilerParams(use_tc_tiling_on_sc=False),
    )
    def agg_kernel(z_hbm, src_hbm, dst_hbm, out_hbm, src_v, dst_v, bufs, acc,
                   *gsem):
        c = lax.axis_index("c")
        s = lax.axis_index("s")
        wid = c * NS + s
        _zero_stripe(bufs, acc, s)
        pltpu.sync_copy(src_hbm.at[wid], src_v)
        pltpu.sync_copy(dst_hbm.at[wid], dst_v)
        plsc.subcore_barrier()
        _edge_pipeline(z_hbm, src_v, dst_v, bufs, gsem, acc, n_blocks)
        plsc.subcore_barrier()
        _drain_stripe(acc, out_hbm, c, s)

    return agg_kernel


def _dinv_of(d0, d1):
    return lax.rsqrt(1.0 + d0[:, :1] + d1[:, :1])


def _mm1_body(x_ref, d0_ref, d1_ref, w_ref, o_ref):
    dinv = _dinv_of(d0_ref[...], d1_ref[...])
    y = jnp.dot(x_ref[...], w_ref[...], preferred_element_type=jnp.float32)
    o_ref[...] = y * dinv


def _fuse_body(a0_ref, a1_ref, z1_ref, d0_ref, d1_ref, b1_ref, w2_ref, o_ref):
    dinv = _dinv_of(d0_ref[...], d1_ref[...])
    agg = jnp.concatenate([a0_ref[...], a1_ref[...]], axis=1)
    h = dinv * (agg + z1_ref[...]) + b1_ref[...]
    h = jnp.maximum(h, 0.0)
    y2 = jnp.dot(h, w2_ref[...], preferred_element_type=jnp.float32)
    o_ref[...] = y2 * dinv


def _fin_body(q0_ref, q1_ref, z2_ref, d0_ref, d1_ref, b2_ref, o_ref):
    dinv = _dinv_of(d0_ref[...], d1_ref[...])
    o_ref[...] = dinv * (q0_ref[...] + q1_ref[...] + z2_ref[...]) + b2_ref[...]


def _row_spec(cols):
    return pl.BlockSpec((BR, cols), lambda i: (i, 0))


def _full_spec(rows, cols):
    return pl.BlockSpec((rows, cols), lambda i: (0, 0))


def kernel(x, edge_index, W1, b1, W2, b2):
    src = edge_index[0].astype(jnp.int32)
    dst = edge_index[1].astype(jnp.int32)
    e = src.shape[0]
    nb2 = -(-e // (NW * K * NBUF)) * NBUF
    nb1 = 2 * nb2
    pad = NW * nb2 * K - e
    src_p = jnp.concatenate([src, jnp.zeros((pad,), jnp.int32)])
    dst_p = jnp.concatenate([dst, jnp.full((pad,), N_NODES, jnp.int32)])
    src_w = src_p.reshape(NW, nb2, K)
    dst_w = dst_p.reshape(NW, nb2, K)
    src_t = src_p.reshape(NS, nb1, K)
    dst_t = dst_p.reshape(NS, nb1, K)

    degp = _make_deg_kernel(nb2)(dst_w)
    d0 = degp[0, :N_NODES]
    d1 = degp[1, :N_NODES]

    grid = (N_NODES // BR,)
    z1 = pl.pallas_call(
        _mm1_body,
        grid=grid,
        in_specs=[_row_spec(IN_DIM), _row_spec(16), _row_spec(16),
                  _full_spec(IN_DIM, HID_DIM)],
        out_specs=_row_spec(HID_DIM),
        out_shape=jax.ShapeDtypeStruct((N_NODES, HID_DIM), jnp.float32),
    )(x, d0, d1, W1)

    half = HID_DIM // 2
    agg1 = _make_agg_split_kernel(nb1, half)(
        z1.reshape(2 * N_NODES, half), src_t, dst_t)

    z2 = pl.pallas_call(
        _fuse_body,
        grid=grid,
        in_specs=[_row_spec(half), _row_spec(half), _row_spec(HID_DIM),
                  _row_spec(16), _row_spec(16),
                  _full_spec(1, HID_DIM), _full_spec(HID_DIM, OUT_DIM)],
        out_specs=_row_spec(OUT_DIM),
        out_shape=jax.ShapeDtypeStruct((N_NODES, OUT_DIM), jnp.float32),
    )(agg1[0, :N_NODES], agg1[1, :N_NODES], z1, d0, d1,
      b1.reshape(1, HID_DIM), W2)

    agg2 = _make_agg_kernel(nb2, OUT_DIM)(z2, src_w, dst_w)

    out = pl.pallas_call(
        _fin_body,
        grid=grid,
        in_specs=[_row_spec(OUT_DIM), _row_spec(OUT_DIM), _row_spec(OUT_DIM),
                  _row_spec(16), _row_spec(16), _full_spec(1, OUT_DIM)],
        out_specs=_row_spec(OUT_DIM),
        out_shape=jax.ShapeDtypeStruct((N_NODES, OUT_DIM), jnp.float32),
    )(agg2[0, :N_NODES], agg2[1, :N_NODES], z2, d0, d1, b2.reshape(1, OUT_DIM))

    return out


# speedup vs baseline: 1.1702x; 1.1702x over previous
import functools

import jax
import jax.numpy as jnp
from jax import lax
from jax.experimental import pallas as pl
from jax.experimental.pallas import tpu as pltpu
from jax.experimental.pallas import tpu_sc as plsc

N_NODES = 10000
IN_DIM = 128
HID_DIM = 128
OUT_DIM = 16

NC = 2
NS = 16
NW = NC * NS
STRIPE = 640
NPAD = NS * STRIPE
BR = 1000


def _fill(ref, rows, cols, value):
    groups = cols // 16

    def body(t, carry):
        ref[t // groups, pl.ds((t % groups) * 16, 16)] = jnp.full(
            (16,), value, jnp.float32)
        return carry

    lax.fori_loop(0, rows * groups, body, 0)


def _zero_stripe(zref, rows, acc, s):
    d = zref.shape[-1]
    groups = d // 16

    def body(t, carry):
        zref[t // groups, pl.ds((t % groups) * 16, 16)] = jnp.zeros(
            (16,), jnp.float32)
        return carry

    lax.fori_loop(0, rows * groups, body, 0)
    for kk in range(STRIPE // rows):
        pltpu.sync_copy(zref, acc.at[pl.ds(s * STRIPE + kk * rows, rows)])


def _drain_stripe(acc, out_hbm, c, s, rows):
    for kk in range(STRIPE // rows):
        off = s * STRIPE + kk * rows
        pltpu.sync_copy(acc.at[pl.ds(off, rows)],
                        out_hbm.at[c, pl.ds(off, rows)])


def _edge_pipeline(z_hbm, src_v, dst_v, bufs, gsem, acc, n_blocks, nbuf):
    for b in range(nbuf):
        pltpu.async_copy(z_hbm.at[src_v.at[b]], bufs.at[b], gsem[b])

    def body(g, carry):
        for b in range(nbuf):
            j = g * nbuf + b
            pltpu.make_async_copy(
                z_hbm.at[src_v.at[j]], bufs.at[b], gsem[b]).wait()
            pltpu.sync_copy(bufs.at[b], acc.at[dst_v.at[j]], add=True)

            @pl.when(j + nbuf < n_blocks)
            def _():
                pltpu.async_copy(
                    z_hbm.at[src_v.at[j + nbuf]], bufs.at[b], gsem[b])
        return carry

    lax.fori_loop(0, n_blocks // nbuf, body, 0)


def _make_deg_kernel(n_blocks, k):
    mesh = plsc.VectorSubcoreMesh(core_axis_name="c", subcore_axis_name="s")

    @functools.partial(
        pl.kernel,
        mesh=mesh,
        out_type=jax.ShapeDtypeStruct((NC, NPAD, 16), jnp.float32),
        scratch_types=[
            pltpu.VMEM((n_blocks, k), jnp.int32),
            pltpu.VMEM((k, 16), jnp.float32),
            pltpu.VMEM_SHARED((NPAD, 16), jnp.float32),
        ],
        compiler_params=pltpu.CompilerParams(use_tc_tiling_on_sc=False),
    )
    def deg_kernel(dst_hbm, out_hbm, dst_v, ones_v, acc):
        c = lax.axis_index("c")
        s = lax.axis_index("s")
        wid = c * NS + s
        _zero_stripe(ones_v, k, acc, s)
        _fill(ones_v, k, 16, 1.0)
        pltpu.sync_copy(dst_hbm.at[wid], dst_v)
        plsc.subcore_barrier()

        def body(j, carry):
            pltpu.sync_copy(ones_v, acc.at[dst_v.at[j]], add=True)
            return carry

        lax.fori_loop(0, n_blocks, body, 0)
        plsc.subcore_barrier()
        _drain_stripe(acc, out_hbm, c, s, k)

    return deg_kernel


def _make_agg_kernel(n_blocks, d, k, nbuf):
    assert n_blocks % nbuf == 0 and STRIPE % k == 0
    mesh = plsc.VectorSubcoreMesh(core_axis_name="c", subcore_axis_name="s")

    @functools.partial(
        pl.kernel,
        mesh=mesh,
        out_type=jax.ShapeDtypeStruct((NC, NPAD, d), jnp.float32),
        scratch_types=[
            pltpu.VMEM((n_blocks, k), jnp.int32),
            pltpu.VMEM((n_blocks, k), jnp.int32),
            pltpu.VMEM((nbuf, k, d), jnp.float32),
            pltpu.VMEM_SHARED((NPAD, d), jnp.float32),
        ] + [pltpu.SemaphoreType.DMA] * nbuf,
        compiler_params=pltpu.CompilerParams(use_tc_tiling_on_sc=False),
    )
    def agg_kernel(z_hbm, src_hbm, dst_hbm, out_hbm, src_v, dst_v, bufs, acc,
                   *gsem):
        c = lax.axis_index("c")
        s = lax.axis_index("s")
        wid = c * NS + s
        _zero_stripe(bufs.at[0], k, acc, s)
        pltpu.sync_copy(src_hbm.at[wid], src_v)
        pltpu.sync_copy(dst_hbm.at[wid], dst_v)
        plsc.subcore_barrier()
        _edge_pipeline(z_hbm, src_v, dst_v, bufs, gsem, acc, n_blocks, nbuf)
        plsc.subcore_barrier()
        _drain_stripe(acc, out_hbm, c, s, k)

    return agg_kernel


def _dinv_of(d0, d1):
    return lax.rsqrt(1.0 + d0[:, :1] + d1[:, :1])


def _mm1_body(x_ref, d0_ref, d1_ref, w_ref, o_ref):
    dinv = _dinv_of(d0_ref[...], d1_ref[...])
    y = jnp.dot(x_ref[...], w_ref[...], preferred_element_type=jnp.float32)
    o_ref[...] = y * dinv


def _fuse_body(p0_ref, p1_ref, z1_ref, d0_ref, d1_ref, b1_ref, w2_ref, o_ref):
    dinv = _dinv_of(d0_ref[...], d1_ref[...])
    h = dinv * (p0_ref[...] + p1_ref[...] + z1_ref[...]) + b1_ref[...]
    h = jnp.maximum(h, 0.0)
    y2 = jnp.dot(h, w2_ref[...], preferred_element_type=jnp.float32)
    o_ref[...] = y2 * dinv


def _fin_body(q0_ref, q1_ref, z2_ref, d0_ref, d1_ref, b2_ref, o_ref):
    dinv = _dinv_of(d0_ref[...], d1_ref[...])
    o_ref[...] = dinv * (q0_ref[...] + q1_ref[...] + z2_ref[...]) + b2_ref[...]


def _row_spec(cols):
    return pl.BlockSpec((BR, cols), lambda i: (i, 0))


def _full_spec(rows, cols):
    return pl.BlockSpec((rows, cols), lambda i: (0, 0))


def _pad_edges(src, dst, n_blocks, k):
    e = src.shape[0]
    pad = NW * n_blocks * k - e
    src_p = jnp.concatenate([src, jnp.zeros((pad,), jnp.int32)])
    dst_p = jnp.concatenate([dst, jnp.full((pad,), N_NODES, jnp.int32)])
    return src_p.reshape(NW, n_blocks, k), dst_p.reshape(NW, n_blocks, k)


def kernel(x, edge_index, W1, b1, W2, b2):
    src = edge_index[0].astype(jnp.int32)
    dst = edge_index[1].astype(jnp.int32)
    e = src.shape[0]

    K1, NBUF1 = 64, 3
    nb1 = -(-e // (NW * K1 * NBUF1)) * NBUF1
    src_1, dst_1 = _pad_edges(src, dst, nb1, K1)
    K2, NBUF2 = 128, 4
    nb2 = -(-e // (NW * K2 * NBUF2)) * NBUF2
    src_2, dst_2 = _pad_edges(src, dst, nb2, K2)

    degp = _make_deg_kernel(nb2, K2)(dst_2)
    d0 = degp[0, :N_NODES]
    d1 = degp[1, :N_NODES]

    grid = (N_NODES // BR,)
    z1 = pl.pallas_call(
        _mm1_body,
        grid=grid,
        in_specs=[_row_spec(IN_DIM), _row_spec(16), _row_spec(16),
                  _full_spec(IN_DIM, HID_DIM)],
        out_specs=_row_spec(HID_DIM),
        out_shape=jax.ShapeDtypeStruct((N_NODES, HID_DIM), jnp.float32),
    )(x, d0, d1, W1)

    agg1 = _make_agg_kernel(nb1, HID_DIM, K1, NBUF1)(z1, src_1, dst_1)

    z2 = pl.pallas_call(
        _fuse_body,
        grid=grid,
        in_specs=[_row_spec(HID_DIM), _row_spec(HID_DIM), _row_spec(HID_DIM),
                  _row_spec(16), _row_spec(16),
                  _full_spec(1, HID_DIM), _full_spec(HID_DIM, OUT_DIM)],
        out_specs=_row_spec(OUT_DIM),
        out_shape=jax.ShapeDtypeStruct((N_NODES, OUT_DIM), jnp.float32),
    )(agg1[0, :N_NODES], agg1[1, :N_NODES], z1, d0, d1,
      b1.reshape(1, HID_DIM), W2)

    agg2 = _make_agg_kernel(nb2, OUT_DIM, K2, NBUF2)(z2, src_2, dst_2)

    out = pl.pallas_call(
        _fin_body,
        grid=grid,
        in_specs=[_row_spec(OUT_DIM), _row_spec(OUT_DIM), _row_spec(OUT_DIM),
                  _row_spec(16), _row_spec(16), _full_spec(1, OUT_DIM)],
        out_specs=_row_spec(OUT_DIM),
        out_shape=jax.ShapeDtypeStruct((N_NODES, OUT_DIM), jnp.float32),
    )(agg2[0, :N_NODES], agg2[1, :N_NODES], z2, d0, d1, b2.reshape(1, OUT_DIM))

    return out


# speedup vs baseline: 1.4413x; 1.2316x over previous
import functools

import jax
import jax.numpy as jnp
from jax import lax
from jax.experimental import pallas as pl
from jax.experimental.pallas import tpu as pltpu
from jax.experimental.pallas import tpu_sc as plsc

N_NODES = 10000
IN_DIM = 128
HID_DIM = 128
OUT_DIM = 16

NC = 2
NS = 16
NW = NC * NS
STRIPE = 640
NPAD = NS * STRIPE
BR = 1000


def _fill(ref, rows, cols, value):
    groups = cols // 16

    def body(t, carry):
        ref[t // groups, pl.ds((t % groups) * 16, 16)] = jnp.full(
            (16,), value, jnp.float32)
        return carry

    lax.fori_loop(0, rows * groups, body, 0)


def _zero_stripe(zref, rows, acc, s):
    d = zref.shape[-1]
    groups = d // 16

    def body(t, carry):
        zref[t // groups, pl.ds((t % groups) * 16, 16)] = jnp.zeros(
            (16,), jnp.float32)
        return carry

    lax.fori_loop(0, rows * groups, body, 0)
    for kk in range(STRIPE // rows):
        pltpu.sync_copy(zref, acc.at[pl.ds(s * STRIPE + kk * rows, rows)])


def _drain_stripe(acc, out_hbm, c, s, rows):
    for kk in range(STRIPE // rows):
        off = s * STRIPE + kk * rows
        pltpu.sync_copy(acc.at[pl.ds(off, rows)],
                        out_hbm.at[c, pl.ds(off, rows)])


def _edge_pipeline(z_hbm, src_v, dst_v, bufs, gsem, acc, n_blocks, nbuf):
    for b in range(nbuf):
        pltpu.async_copy(z_hbm.at[src_v.at[b]], bufs.at[b], gsem[b])

    def body(g, carry):
        for b in range(nbuf):
            j = g * nbuf + b
            pltpu.make_async_copy(
                z_hbm.at[src_v.at[j]], bufs.at[b], gsem[b]).wait()
            pltpu.sync_copy(bufs.at[b], acc.at[dst_v.at[j]], add=True)

            @pl.when(j + nbuf < n_blocks)
            def _():
                pltpu.async_copy(
                    z_hbm.at[src_v.at[j + nbuf]], bufs.at[b], gsem[b])
        return carry

    lax.fori_loop(0, n_blocks // nbuf, body, 0)


def _make_deg_kernel(n_blocks, k):
    mesh = plsc.VectorSubcoreMesh(core_axis_name="c", subcore_axis_name="s")

    @functools.partial(
        pl.kernel,
        mesh=mesh,
        out_type=jax.ShapeDtypeStruct((NC, NPAD, 16), jnp.float32),
        scratch_types=[
            pltpu.VMEM((n_blocks, k), jnp.int32),
            pltpu.VMEM((k, 16), jnp.float32),
            pltpu.VMEM_SHARED((NPAD, 16), jnp.float32),
        ],
        compiler_params=pltpu.CompilerParams(use_tc_tiling_on_sc=False),
    )
    def deg_kernel(dst_hbm, out_hbm, dst_v, ones_v, acc):
        c = lax.axis_index("c")
        s = lax.axis_index("s")
        wid = c * NS + s
        _zero_stripe(ones_v, k, acc, s)
        _fill(ones_v, k, 16, 1.0)
        pltpu.sync_copy(dst_hbm.at[wid], dst_v)
        plsc.subcore_barrier()

        def body(j, carry):
            pltpu.sync_copy(ones_v, acc.at[dst_v.at[j]], add=True)
            return carry

        lax.fori_loop(0, n_blocks, body, 0)
        plsc.subcore_barrier()
        _drain_stripe(acc, out_hbm, c, s, k)

    return deg_kernel


SLOW_CORE = 1


def _make_agg_kernel(nb_fast, nb_slow, d, k, nbuf):
    assert nb_fast % nbuf == 0 and nb_slow % nbuf == 0 and STRIPE % k == 0
    nb_max = max(nb_fast, nb_slow)
    mesh = plsc.VectorSubcoreMesh(core_axis_name="c", subcore_axis_name="s")

    @functools.partial(
        pl.kernel,
        mesh=mesh,
        out_type=jax.ShapeDtypeStruct((NC, NPAD, d), jnp.float32),
        scratch_types=[
            pltpu.VMEM((nb_max, k), jnp.int32),
            pltpu.VMEM((nb_max, k), jnp.int32),
            pltpu.VMEM((nbuf, k, d), jnp.float32),
            pltpu.VMEM_SHARED((NPAD, d), jnp.float32),
        ] + [pltpu.SemaphoreType.DMA] * nbuf,
        compiler_params=pltpu.CompilerParams(use_tc_tiling_on_sc=False),
    )
    def agg_kernel(z_hbm, src_hbm, dst_hbm, out_hbm, src_v, dst_v, bufs, acc,
                   *gsem):
        c = lax.axis_index("c")
        s = lax.axis_index("s")
        wid = c * NS + s
        n_my = jnp.where(c == SLOW_CORE, nb_slow, nb_fast)
        _zero_stripe(bufs.at[0], k, acc, s)
        pltpu.sync_copy(src_hbm.at[wid], src_v)
        pltpu.sync_copy(dst_hbm.at[wid], dst_v)
        plsc.subcore_barrier()
        _edge_pipeline(z_hbm, src_v, dst_v, bufs, gsem, acc, n_my, nbuf)
        plsc.subcore_barrier()
        _drain_stripe(acc, out_hbm, c, s, k)

    return agg_kernel


def _dinv_of(d0, d1):
    return lax.rsqrt(1.0 + d0[:, :1] + d1[:, :1])


def _mm1_body(x_ref, d0_ref, d1_ref, w_ref, o_ref):
    dinv = _dinv_of(d0_ref[...], d1_ref[...])
    y = jnp.dot(x_ref[...], w_ref[...], preferred_element_type=jnp.float32)
    o_ref[...] = y * dinv


def _fuse_body(p0_ref, p1_ref, z1_ref, d0_ref, d1_ref, b1_ref, w2_ref, o_ref):
    dinv = _dinv_of(d0_ref[...], d1_ref[...])
    h = dinv * (p0_ref[...] + p1_ref[...] + z1_ref[...]) + b1_ref[...]
    h = jnp.maximum(h, 0.0)
    y2 = jnp.dot(h, w2_ref[...], preferred_element_type=jnp.float32)
    o_ref[...] = y2 * dinv


def _fin_body(q0_ref, q1_ref, z2_ref, d0_ref, d1_ref, b2_ref, o_ref):
    dinv = _dinv_of(d0_ref[...], d1_ref[...])
    o_ref[...] = dinv * (q0_ref[...] + q1_ref[...] + z2_ref[...]) + b2_ref[...]


def _row_spec(cols):
    return pl.BlockSpec((BR, cols), lambda i: (i, 0))


def _full_spec(rows, cols):
    return pl.BlockSpec((rows, cols), lambda i: (0, 0))


def _pad_edges(src, dst, n_blocks, k):
    e = src.shape[0]
    pad = NW * n_blocks * k - e
    src_p = jnp.concatenate([src, jnp.zeros((pad,), jnp.int32)])
    dst_p = jnp.concatenate([dst, jnp.full((pad,), N_NODES, jnp.int32)])
    return src_p.reshape(NW, n_blocks, k), dst_p.reshape(NW, n_blocks, k)


def _pad_edges_split(src, dst, nb_fast, nb_slow, k):
    e = src.shape[0]
    nb_max = max(nb_fast, nb_slow)
    slots = NS * (nb_fast + nb_slow) * k
    pad = slots - e

    def build(a, fill):
        a_p = jnp.concatenate([a, jnp.full((pad,), fill, jnp.int32)])
        cut = NS * nb_fast * k
        fast = a_p[:cut].reshape(NS, nb_fast, k)
        slow = a_p[cut:].reshape(NS, nb_slow, k)
        fast = jnp.pad(fast, ((0, 0), (0, nb_max - nb_fast), (0, 0)))
        slow = jnp.pad(slow, ((0, 0), (0, nb_max - nb_slow), (0, 0)))
        both = [fast, slow] if SLOW_CORE == 1 else [slow, fast]
        return jnp.concatenate(both, axis=0)

    return build(src, 0), build(dst, N_NODES)


def kernel(x, edge_index, W1, b1, W2, b2):
    src = edge_index[0].astype(jnp.int32)
    dst = edge_index[1].astype(jnp.int32)
    e = src.shape[0]

    K1, NBUF1 = 64, 2
    NB1_FAST, NB1_SLOW = 236, 78
    src_1, dst_1 = _pad_edges_split(src, dst, NB1_FAST, NB1_SLOW, K1)
    K2, NBUF2 = 128, 4
    NB2_FAST, NB2_SLOW = 120, 40
    src_2, dst_2 = _pad_edges_split(src, dst, NB2_FAST, NB2_SLOW, K2)
    nbd = -(-e // (NW * K2))
    _, dst_d = _pad_edges(src, dst, nbd, K2)

    degp = _make_deg_kernel(nbd, K2)(dst_d)
    d0 = degp[0, :N_NODES]
    d1 = degp[1, :N_NODES]

    grid = (N_NODES // BR,)
    z1 = pl.pallas_call(
        _mm1_body,
        grid=grid,
        in_specs=[_row_spec(IN_DIM), _row_spec(16), _row_spec(16),
                  _full_spec(IN_DIM, HID_DIM)],
        out_specs=_row_spec(HID_DIM),
        out_shape=jax.ShapeDtypeStruct((N_NODES, HID_DIM), jnp.float32),
    )(x, d0, d1, W1)

    agg1 = _make_agg_kernel(NB1_FAST, NB1_SLOW, HID_DIM, K1, NBUF1)(
        z1, src_1, dst_1)

    z2 = pl.pallas_call(
        _fuse_body,
        grid=grid,
        in_specs=[_row_spec(HID_DIM), _row_spec(HID_DIM), _row_spec(HID_DIM),
                  _row_spec(16), _row_spec(16),
                  _full_spec(1, HID_DIM), _full_spec(HID_DIM, OUT_DIM)],
        out_specs=_row_spec(OUT_DIM),
        out_shape=jax.ShapeDtypeStruct((N_NODES, OUT_DIM), jnp.float32),
    )(agg1[0, :N_NODES], agg1[1, :N_NODES], z1, d0, d1,
      b1.reshape(1, HID_DIM), W2)

    agg2 = _make_agg_kernel(NB2_FAST, NB2_SLOW, OUT_DIM, K2, NBUF2)(
        z2, src_2, dst_2)

    out = pl.pallas_call(
        _fin_body,
        grid=grid,
        in_specs=[_row_spec(OUT_DIM), _row_spec(OUT_DIM), _row_spec(OUT_DIM),
                  _row_spec(16), _row_spec(16), _full_spec(1, OUT_DIM)],
        out_specs=_row_spec(OUT_DIM),
        out_shape=jax.ShapeDtypeStruct((N_NODES, OUT_DIM), jnp.float32),
    )(agg2[0, :N_NODES], agg2[1, :N_NODES], z2, d0, d1, b2.reshape(1, OUT_DIM))

    return out
